# baseline (speedup 1.0000x reference)
import functools

import jax
import jax.numpy as jnp
from jax import lax
from jax.experimental import pallas as pl
from jax.experimental.pallas import tpu as pltpu
from jax.experimental.pallas import tpu_sc as plsc

B, S, D = 4, 2048, 1024
HALF = D // 2
T = B * S
NC, NS = 2, 16
NW = NC * NS
PER_W = T // NW
CHUNK = 32
NCHUNK = PER_W // CHUNK
LANES = 16
VPH = HALF // LANES

_mesh = plsc.VectorSubcoreMesh(
    core_axis_name="c", subcore_axis_name="s", num_cores=NC, num_subcores=NS)


@functools.partial(
    pl.kernel,
    out_type=jax.ShapeDtypeStruct((T, D), jnp.float32),
    mesh=_mesh,
    scratch_types=[
        pltpu.VMEM((CHUNK,), jnp.int32),
        pltpu.VMEM((CHUNK,), jnp.int32),
        pltpu.VMEM((CHUNK, D), jnp.float32),
        pltpu.VMEM((CHUNK, HALF), jnp.float32),
        pltpu.VMEM((CHUNK, HALF), jnp.float32),
        pltpu.SemaphoreType.DMA,
    ],
)
def _pos_emb_add(x_hbm, idx0_hbm, idx1_hbm, pe1_hbm, pe2_hbm, out_hbm,
                 idx0_v, idx1_v, x_v, g1_v, g2_v, sem):
    wid = lax.axis_index("s") * NC + lax.axis_index("c")
    base = wid * PER_W
    for c in range(NCHUNK):
        off = base + c * CHUNK
        pltpu.sync_copy(idx0_hbm.at[pl.ds(off, CHUNK)], idx0_v)
        pltpu.sync_copy(idx1_hbm.at[pl.ds(off, CHUNK)], idx1_v)
        cp_x = pltpu.async_copy(x_hbm.at[pl.ds(off, CHUNK)], x_v, sem)
        cp_g1 = pltpu.async_copy(pe1_hbm.at[idx0_v], g1_v, sem)
        cp_g2 = pltpu.async_copy(pe2_hbm.at[idx1_v], g2_v, sem)
        cp_x.wait()
        cp_g1.wait()
        cp_g2.wait()

        def add_row(k, _):
            for j in range(VPH):
                plsc.addupdate(x_v.at[k, pl.ds(j * LANES, LANES)],
                               g1_v[k, pl.ds(j * LANES, LANES)])
                plsc.addupdate(x_v.at[k, pl.ds(HALF + j * LANES, LANES)],
                               g2_v[k, pl.ds(j * LANES, LANES)])
            return _

        lax.fori_loop(0, CHUNK, add_row, 0)
        pltpu.sync_copy(x_v, out_hbm.at[pl.ds(off, CHUNK)])


def kernel(inputs, positions, pe1, pe2):
    pos = positions.astype(jnp.int32).reshape(T, 2)
    out = _pos_emb_add(inputs.reshape(T, D), pos[:, 0], pos[:, 1], pe1, pe2)
    return out.reshape(B, S, D)


# speedup vs baseline: 1.0076x; 1.0076x over previous
import functools

import jax
import jax.numpy as jnp
from jax import lax
from jax.experimental import pallas as pl
from jax.experimental.pallas import tpu as pltpu
from jax.experimental.pallas import tpu_sc as plsc

B, S, D = 4, 2048, 1024
HALF = D // 2
T = B * S
NC, NS = 2, 16
NW = NC * NS
PER_W = T // NW
CHUNK = 32
NCHUNK = PER_W // CHUNK
GIDX = 2 * CHUNK
LANES = 16
VPH = HALF // LANES

_mesh = plsc.VectorSubcoreMesh(
    core_axis_name="c", subcore_axis_name="s", num_cores=NC, num_subcores=NS)


@functools.partial(
    pl.kernel,
    out_type=jax.ShapeDtypeStruct((T, D), jnp.float32),
    mesh=_mesh,
    scratch_types=[
        pltpu.VMEM((GIDX,), jnp.int32),
        pltpu.VMEM((CHUNK, D), jnp.float32),
        pltpu.VMEM((GIDX, HALF), jnp.float32),
        pltpu.SemaphoreType.DMA,
    ],
)
def _pos_emb_add(x_hbm, idxc_hbm, pec_hbm, out_hbm, idx_v, x_v, g_v, sem):
    wid = lax.axis_index("s") * NC + lax.axis_index("c")
    base = wid * PER_W
    for c in range(NCHUNK):
        off = base + c * CHUNK
        pltpu.sync_copy(idxc_hbm.at[pl.ds(2 * off, GIDX)], idx_v)
        cp_x = pltpu.async_copy(x_hbm.at[pl.ds(off, CHUNK)], x_v, sem)
        cp_g = pltpu.async_copy(pec_hbm.at[idx_v], g_v, sem)
        cp_x.wait()
        cp_g.wait()

        def add_row(k, _):
            for j in range(VPH):
                plsc.addupdate(x_v.at[k, pl.ds(j * LANES, LANES)],
                               g_v[k, pl.ds(j * LANES, LANES)])
                plsc.addupdate(x_v.at[k, pl.ds(HALF + j * LANES, LANES)],
                               g_v[CHUNK + k, pl.ds(j * LANES, LANES)])
            return _

        lax.fori_loop(0, CHUNK, add_row, 0)
        pltpu.sync_copy(x_v, out_hbm.at[pl.ds(off, CHUNK)])


def kernel(inputs, positions, pe1, pe2):
    pos = (positions.astype(jnp.int32)
           + jnp.array([0, pe1.shape[0]], jnp.int32))
    idxc = pos.reshape(T // CHUNK, CHUNK, 2).transpose(0, 2, 1).reshape(2 * T)
    pec = jnp.concatenate([pe1, pe2], axis=0)
    out = _pos_emb_add(inputs.reshape(T, D), idxc, pec)
    return out.reshape(B, S, D)


# speedup vs baseline: 1.2401x; 1.2307x over previous
import functools

import jax
import jax.numpy as jnp
from jax import lax
from jax.experimental import pallas as pl
from jax.experimental.pallas import tpu as pltpu
from jax.experimental.pallas import tpu_sc as plsc

B, S, D = 4, 2048, 1024
HALF = D // 2
T = B * S
NC, NS = 2, 16
NW = NC * NS
PER_W = T // NW
CHUNK = 16
NCHUNK = PER_W // CHUNK
GIDX = 2 * CHUNK
NBUF = 3
LANES = 16
VPH = HALF // LANES

_mesh = plsc.VectorSubcoreMesh(
    core_axis_name="c", subcore_axis_name="s", num_cores=NC, num_subcores=NS)


@functools.partial(
    pl.kernel,
    out_type=jax.ShapeDtypeStruct((T, D), jnp.float32),
    mesh=_mesh,
    scratch_types=[
        [pltpu.VMEM((GIDX,), jnp.int32) for _ in range(NBUF)],
        [pltpu.VMEM((CHUNK, D), jnp.float32) for _ in range(NBUF)],
        [pltpu.VMEM((GIDX, HALF), jnp.float32) for _ in range(NBUF)],
        [pltpu.SemaphoreType.DMA for _ in range(NBUF)],
        [pltpu.SemaphoreType.DMA for _ in range(NBUF)],
    ],
)
def _pos_emb_add(x_hbm, idxc_hbm, pec_hbm, out_hbm,
                 idx_v, x_v, g_v, sem_in, sem_out):
    wid = lax.axis_index("s") * NC + lax.axis_index("c")
    base = wid * PER_W

    def issue_in(c):
        s = c % NBUF
        off = base + c * CHUNK
        pltpu.sync_copy(idxc_hbm.at[pl.ds(2 * off, GIDX)], idx_v[s])
        pltpu.async_copy(x_hbm.at[pl.ds(off, CHUNK)], x_v[s], sem_in[s])
        pltpu.async_copy(pec_hbm.at[idx_v[s]], g_v[s], sem_in[s])

    issue_in(0)
    issue_in(1)

    for c in range(NCHUNK):
        s = c % NBUF
        off = base + c * CHUNK
        pltpu.make_async_copy(x_hbm.at[pl.ds(off, CHUNK)], x_v[s],
                              sem_in[s]).wait()
        pltpu.make_async_copy(pec_hbm.at[idx_v[s]], g_v[s], sem_in[s]).wait()

        def add_row(k, _):
            for j in range(VPH):
                plsc.addupdate(x_v[s].at[k, pl.ds(j * LANES, LANES)],
                               g_v[s][k, pl.ds(j * LANES, LANES)])
                plsc.addupdate(x_v[s].at[k, pl.ds(HALF + j * LANES, LANES)],
                               g_v[s][CHUNK + k, pl.ds(j * LANES, LANES)])
            return _

        lax.fori_loop(0, CHUNK, add_row, 0)

        if c + 2 < NCHUNK:
            if c >= 1:
                sp = (c - 1) % NBUF
                offp = base + (c - 1) * CHUNK
                pltpu.make_async_copy(
                    x_v[sp], out_hbm.at[pl.ds(offp, CHUNK)],
                    sem_out[sp]).wait()
            issue_in(c + 2)

        pltpu.async_copy(x_v[s], out_hbm.at[pl.ds(off, CHUNK)], sem_out[s])

    for c in range(NCHUNK - NBUF, NCHUNK):
        s = c % NBUF
        off = base + c * CHUNK
        pltpu.make_async_copy(x_v[s], out_hbm.at[pl.ds(off, CHUNK)],
                              sem_out[s]).wait()


def kernel(inputs, positions, pe1, pe2):
    pos = (positions.astype(jnp.int32)
           + jnp.array([0, pe1.shape[0]], jnp.int32))
    idxc = pos.reshape(T // CHUNK, CHUNK, 2).transpose(0, 2, 1).reshape(2 * T)
    pec = jnp.concatenate([pe1, pe2], axis=0)
    out = _pos_emb_add(inputs.reshape(T, D), idxc, pec)
    return out.reshape(B, S, D)


# speedup vs baseline: 1.3357x; 1.0771x over previous
import functools

import jax
import jax.numpy as jnp
from jax import lax
from jax.experimental import pallas as pl
from jax.experimental.pallas import tpu as pltpu
from jax.experimental.pallas import tpu_sc as plsc

B, S, D = 4, 2048, 1024
HALF = D // 2
T = B * S
NC, NS = 2, 16
NW = NC * NS
PER_W = T // NW
CHUNK = 16
NCHUNK = PER_W // CHUNK
GIDX = 2 * CHUNK
NBUF = 3
LANES = 16
VPH = HALF // LANES

_mesh = plsc.VectorSubcoreMesh(
    core_axis_name="c", subcore_axis_name="s", num_cores=NC, num_subcores=NS)


@functools.partial(
    pl.kernel,
    out_type=jax.ShapeDtypeStruct((T, D), jnp.float32),
    mesh=_mesh,
    scratch_types=[
        pltpu.VMEM((2 * PER_W,), jnp.int32),
        [pltpu.VMEM((CHUNK, D), jnp.float32) for _ in range(NBUF)],
        [pltpu.VMEM((GIDX, HALF), jnp.float32) for _ in range(NBUF)],
        [pltpu.SemaphoreType.DMA for _ in range(NBUF)],
        [pltpu.SemaphoreType.DMA for _ in range(NBUF)],
    ],
)
def _pos_emb_add(x_hbm, idxc_hbm, pec_hbm, out_hbm,
                 idx_v, x_v, g_v, sem_in, sem_out):
    wid = lax.axis_index("s") * NC + lax.axis_index("c")
    base = wid * PER_W
    pltpu.sync_copy(idxc_hbm.at[pl.ds(2 * base, 2 * PER_W)], idx_v)

    def gather_copy(c):
        s = c % NBUF
        return pltpu.make_async_copy(
            pec_hbm.at[idx_v.at[pl.ds(c * GIDX, GIDX)]], g_v[s], sem_in[s])

    def issue_in(c):
        s = c % NBUF
        off = base + c * CHUNK
        pltpu.async_copy(x_hbm.at[pl.ds(off, CHUNK)], x_v[s], sem_in[s])
        gather_copy(c).start()

    issue_in(0)
    issue_in(1)

    for c in range(NCHUNK):
        s = c % NBUF
        off = base + c * CHUNK
        pltpu.make_async_copy(x_hbm.at[pl.ds(off, CHUNK)], x_v[s],
                              sem_in[s]).wait()
        gather_copy(c).wait()

        def add_row(k, _):
            for j in range(VPH):
                plsc.addupdate(x_v[s].at[k, pl.ds(j * LANES, LANES)],
                               g_v[s][k, pl.ds(j * LANES, LANES)])
                plsc.addupdate(x_v[s].at[k, pl.ds(HALF + j * LANES, LANES)],
                               g_v[s][CHUNK + k, pl.ds(j * LANES, LANES)])
            return _

        lax.fori_loop(0, CHUNK, add_row, 0)

        if c + 2 < NCHUNK:
            if c >= 1:
                sp = (c - 1) % NBUF
                offp = base + (c - 1) * CHUNK
                pltpu.make_async_copy(
                    x_v[sp], out_hbm.at[pl.ds(offp, CHUNK)],
                    sem_out[sp]).wait()
            issue_in(c + 2)

        pltpu.async_copy(x_v[s], out_hbm.at[pl.ds(off, CHUNK)], sem_out[s])

    for c in range(NCHUNK - NBUF, NCHUNK):
        s = c % NBUF
        off = base + c * CHUNK
        pltpu.make_async_copy(x_v[s], out_hbm.at[pl.ds(off, CHUNK)],
                              sem_out[s]).wait()


def kernel(inputs, positions, pe1, pe2):
    pos = (positions.astype(jnp.int32)
           + jnp.array([0, pe1.shape[0]], jnp.int32))
    idxc = pos.reshape(T // CHUNK, CHUNK, 2).transpose(0, 2, 1).reshape(2 * T)
    pec = jnp.concatenate([pe1, pe2], axis=0)
    out = _pos_emb_add(inputs.reshape(T, D), idxc, pec)
    return out.reshape(B, S, D)


# speedup vs baseline: 1.3755x; 1.0298x over previous
import functools

import jax
import jax.numpy as jnp
from jax import lax
from jax.experimental import pallas as pl
from jax.experimental.pallas import tpu as pltpu
from jax.experimental.pallas import tpu_sc as plsc

B, S, D = 4, 2048, 1024
HALF = D // 2
T = B * S
NC, NS = 2, 16
NW = NC * NS
PER_W = T // NW
CHUNK = 16
NCHUNK = PER_W // CHUNK
NBUF = 3
LANES = 16
VPH = HALF // LANES

_mesh = plsc.VectorSubcoreMesh(
    core_axis_name="c", subcore_axis_name="s", num_cores=NC, num_subcores=NS)


@functools.partial(
    pl.kernel,
    out_type=jax.ShapeDtypeStruct((T, D), jnp.float32),
    mesh=_mesh,
    scratch_types=[
        pltpu.VMEM((PER_W,), jnp.int32),
        pltpu.VMEM((PER_W,), jnp.int32),
        [pltpu.VMEM((CHUNK, D), jnp.float32) for _ in range(NBUF)],
        [pltpu.VMEM((CHUNK, HALF), jnp.float32) for _ in range(NBUF)],
        [pltpu.VMEM((CHUNK, HALF), jnp.float32) for _ in range(NBUF)],
        [pltpu.SemaphoreType.DMA for _ in range(NBUF)],
        [pltpu.SemaphoreType.DMA for _ in range(NBUF)],
    ],
)
def _pos_emb_add(x_hbm, idx1_hbm, idx2_hbm, pe1_hbm, pe2_hbm, out_hbm,
                 idx1_v, idx2_v, x_v, g1_v, g2_v, sem_in, sem_out):
    wid = lax.axis_index("s") * NC + lax.axis_index("c")
    base = wid * PER_W
    pltpu.sync_copy(idx1_hbm.at[pl.ds(base, PER_W)], idx1_v)
    pltpu.sync_copy(idx2_hbm.at[pl.ds(base, PER_W)], idx2_v)

    def copies(c):
        s = c % NBUF
        off = base + c * CHUNK
        return (
            pltpu.make_async_copy(x_hbm.at[pl.ds(off, CHUNK)], x_v[s],
                                  sem_in[s]),
            pltpu.make_async_copy(
                pe1_hbm.at[idx1_v.at[pl.ds(c * CHUNK, CHUNK)]], g1_v[s],
                sem_in[s]),
            pltpu.make_async_copy(
                pe2_hbm.at[idx2_v.at[pl.ds(c * CHUNK, CHUNK)]], g2_v[s],
                sem_in[s]),
        )

    def issue_in(c):
        for cp in copies(c):
            cp.start()

    issue_in(0)
    issue_in(1)

    for c in range(NCHUNK):
        s = c % NBUF
        off = base + c * CHUNK
        for cp in copies(c):
            cp.wait()

        def add_row(k, _):
            for j in range(VPH):
                plsc.addupdate(x_v[s].at[k, pl.ds(j * LANES, LANES)],
                               g1_v[s][k, pl.ds(j * LANES, LANES)])
                plsc.addupdate(x_v[s].at[k, pl.ds(HALF + j * LANES, LANES)],
                               g2_v[s][k, pl.ds(j * LANES, LANES)])
            return _

        lax.fori_loop(0, CHUNK, add_row, 0)

        if c + 2 < NCHUNK:
            if c >= 1:
                sp = (c - 1) % NBUF
                offp = base + (c - 1) * CHUNK
                pltpu.make_async_copy(
                    x_v[sp], out_hbm.at[pl.ds(offp, CHUNK)],
                    sem_out[sp]).wait()
            issue_in(c + 2)

        pltpu.async_copy(x_v[s], out_hbm.at[pl.ds(off, CHUNK)], sem_out[s])

    for c in range(NCHUNK - NBUF, NCHUNK):
        s = c % NBUF
        off = base + c * CHUNK
        pltpu.make_async_copy(x_v[s], out_hbm.at[pl.ds(off, CHUNK)],
                              sem_out[s]).wait()


def kernel(inputs, positions, pe1, pe2):
    pos = positions.astype(jnp.int32)
    idx1 = pos[:, :, 0].reshape(T)
    idx2 = pos[:, :, 1].reshape(T)
    out = _pos_emb_add(inputs.reshape(T, D), idx1, idx2, pe1, pe2)
    return out.reshape(B, S, D)


# speedup vs baseline: 1.3863x; 1.0079x over previous
import functools

import jax
import jax.numpy as jnp
from jax import lax
from jax.experimental import pallas as pl
from jax.experimental.pallas import tpu as pltpu
from jax.experimental.pallas import tpu_sc as plsc

B, S, D = 4, 2048, 1024
HALF = D // 2
T = B * S
NC, NS = 2, 16
NW = NC * NS
PER_W = T // NW
CHUNK = 16
NCHUNK = PER_W // CHUNK
NBUF = 3
LANES = 16
VPH = HALF // LANES

_mesh = plsc.VectorSubcoreMesh(
    core_axis_name="c", subcore_axis_name="s", num_cores=NC, num_subcores=NS)


@functools.partial(
    pl.kernel,
    out_type=jax.ShapeDtypeStruct((T, D), jnp.float32),
    mesh=_mesh,
    scratch_types=[
        pltpu.VMEM((PER_W,), jnp.int32),
        pltpu.VMEM((PER_W,), jnp.int32),
        [pltpu.VMEM((CHUNK, D), jnp.float32) for _ in range(NBUF)],
        [pltpu.VMEM((CHUNK, HALF), jnp.float32) for _ in range(NBUF)],
        [pltpu.VMEM((CHUNK, HALF), jnp.float32) for _ in range(NBUF)],
        [pltpu.SemaphoreType.DMA for _ in range(NBUF)],
        [pltpu.SemaphoreType.DMA for _ in range(NBUF)],
    ],
)
def _pos_emb_add(x_hbm, idx1_hbm, idx2_hbm, pe1_hbm, pe2_hbm, out_hbm,
                 idx1_v, idx2_v, x_v, g1_v, g2_v, sem_in, sem_out):
    wid = lax.axis_index("s") * NC + lax.axis_index("c")
    base = wid * PER_W
    pltpu.sync_copy(idx1_hbm.at[pl.ds(base, PER_W)], idx1_v)
    pltpu.sync_copy(idx2_hbm.at[pl.ds(base, PER_W)], idx2_v)

    def copies(c):
        s = c % NBUF
        off = base + c * CHUNK
        return (
            pltpu.make_async_copy(x_hbm.at[pl.ds(off, CHUNK)], x_v[s],
                                  sem_in[s]),
            pltpu.make_async_copy(
                pe1_hbm.at[idx1_v.at[pl.ds(c * CHUNK, CHUNK)]], g1_v[s],
                sem_in[s]),
            pltpu.make_async_copy(
                pe2_hbm.at[idx2_v.at[pl.ds(c * CHUNK, CHUNK)]], g2_v[s],
                sem_in[s]),
        )

    def issue_in(c):
        for cp in copies(c):
            cp.start()

    issue_in(0)
    issue_in(1)

    for c in range(NCHUNK):
        s = c % NBUF
        off = base + c * CHUNK
        for cp in copies(c):
            cp.wait()

        def add_row(k, _):
            for j in range(VPH):
                plsc.addupdate(x_v[s].at[k, pl.ds(j * LANES, LANES)],
                               g1_v[s][k, pl.ds(j * LANES, LANES)])
                plsc.addupdate(x_v[s].at[k, pl.ds(HALF + j * LANES, LANES)],
                               g2_v[s][k, pl.ds(j * LANES, LANES)])
            return _

        lax.fori_loop(0, CHUNK, add_row, 0)
        pltpu.async_copy(x_v[s], out_hbm.at[pl.ds(off, CHUNK)], sem_out[s])

        if c + 2 < NCHUNK:
            if c >= 1:
                sp = (c - 1) % NBUF
                offp = base + (c - 1) * CHUNK
                pltpu.make_async_copy(
                    x_v[sp], out_hbm.at[pl.ds(offp, CHUNK)],
                    sem_out[sp]).wait()
            issue_in(c + 2)

    for c in range(NCHUNK - NBUF, NCHUNK):
        s = c % NBUF
        off = base + c * CHUNK
        pltpu.make_async_copy(x_v[s], out_hbm.at[pl.ds(off, CHUNK)],
                              sem_out[s]).wait()


def kernel(inputs, positions, pe1, pe2):
    pos = positions.astype(jnp.int32)
    idx1 = pos[:, :, 0].reshape(T)
    idx2 = pos[:, :, 1].reshape(T)
    out = _pos_emb_add(inputs.reshape(T, D), idx1, idx2, pe1, pe2)
    return out.reshape(B, S, D)
